# baseline (speedup 1.0000x reference)
import functools

import jax
import jax.numpy as jnp
from jax import lax
from jax.experimental import pallas as pl
from jax.experimental.pallas import tpu as pltpu
from jax.experimental.pallas import tpu_sc as plsc

D_MODEL = 768
MAX_LEN = 8192
B = 4
L = 2048

NC = 2
NS = 16
NW = NC * NS
N_GATHER = B * L
ROWS_PER_TILE = N_GATHER // NW
CHUNK = 32
NCHUNK = ROWS_PER_TILE // CHUNK
LANES = 16
VECS_PER_ROW = D_MODEL // LANES

_MESH = plsc.VectorSubcoreMesh(core_axis_name="c", subcore_axis_name="s")


@functools.partial(
    pl.kernel,
    out_type=jax.ShapeDtypeStruct((N_GATHER + B, D_MODEL), jnp.float32),
    mesh=_MESH,
    scratch_types=[
        pltpu.VMEM((NCHUNK, CHUNK), jnp.int32),
        pltpu.VMEM((CHUNK, D_MODEL), jnp.float32),
        pltpu.VMEM((CHUNK, D_MODEL), jnp.float32),
        pltpu.SemaphoreType.DMA,
        pltpu.SemaphoreType.DMA,
    ],
    compiler_params=pltpu.CompilerParams(use_tc_tiling_on_sc=False),
)
def _pe_add(x_hbm, ts_hbm, pe_hbm, out_hbm, idx_v, xbuf, pebuf, sem_x, sem_pe):
    wid = lax.axis_index("s") * NC + lax.axis_index("c")
    base = wid * ROWS_PER_TILE

    pltpu.sync_copy(ts_hbm.at[wid], idx_v)
    for c in range(NCHUNK):
        for j in range(CHUNK // LANES):
            sl = pl.ds(j * LANES, LANES)
            v = idx_v[c, sl]
            idx_v[c, sl] = jnp.minimum(jnp.maximum(v, 0), MAX_LEN - 1)

    @pl.when(wid == 0)
    def _():
        pltpu.sync_copy(x_hbm.at[pl.ds(0, B)], xbuf.at[pl.ds(0, B)])
        pltpu.sync_copy(xbuf.at[pl.ds(0, B)], out_hbm.at[pl.ds(0, B)])

    def chunk_body(c, carry):
        row0 = B + base + c * CHUNK
        cp_x = pltpu.async_copy(x_hbm.at[pl.ds(row0, CHUNK)], xbuf, sem_x)
        cp_pe = pltpu.async_copy(pe_hbm.at[idx_v.at[c]], pebuf, sem_pe)
        cp_x.wait()
        cp_pe.wait()

        def row_body(r, carry2):
            for j in range(VECS_PER_ROW):
                sl = pl.ds(j * LANES, LANES)
                xbuf[r, sl] = xbuf[r, sl] + pebuf[r, sl]
            return carry2

        lax.fori_loop(0, CHUNK, row_body, 0)
        pltpu.sync_copy(xbuf, out_hbm.at[pl.ds(row0, CHUNK)])
        return carry

    lax.fori_loop(0, NCHUNK, chunk_body, 0)


def kernel(x, timestamps, pe):
    x2 = x.reshape((L + 1) * B, D_MODEL)
    ts3 = timestamps.T.reshape(NW, NCHUNK, CHUNK)
    pe2 = pe.reshape(MAX_LEN, D_MODEL)
    out = _pe_add(x2, ts3, pe2)
    return out.reshape(L + 1, B, D_MODEL)


# speedup vs baseline: 1.0656x; 1.0656x over previous
import functools

import jax
import jax.numpy as jnp
from jax import lax
from jax.experimental import pallas as pl
from jax.experimental.pallas import tpu as pltpu
from jax.experimental.pallas import tpu_sc as plsc

D_MODEL = 768
MAX_LEN = 8192
B = 4
L = 2048

NC = 2
NS = 16
NW = NC * NS
N_GATHER = B * L
ROWS_PER_TILE = N_GATHER // NW
CHUNK = 32
NCHUNK = ROWS_PER_TILE // CHUNK
LANES = 16
VECS_PER_ROW = D_MODEL // LANES

_MESH = plsc.VectorSubcoreMesh(core_axis_name="c", subcore_axis_name="s")


@functools.partial(
    pl.kernel,
    out_type=jax.ShapeDtypeStruct((N_GATHER + B, D_MODEL), jnp.float32),
    mesh=_MESH,
    scratch_types=[
        pltpu.VMEM((NCHUNK, CHUNK), jnp.int32),
        pltpu.VMEM((CHUNK, D_MODEL), jnp.float32),
        pltpu.VMEM((CHUNK, D_MODEL), jnp.float32),
        pltpu.SemaphoreType.DMA,
        pltpu.SemaphoreType.DMA,
    ],
    compiler_params=pltpu.CompilerParams(use_tc_tiling_on_sc=False),
)
def _pe_add(x_hbm, ts_hbm, pe_hbm, out_hbm, idx_v, xbuf, pebuf, sem_x, sem_pe):
    wid = lax.axis_index("s") * NC + lax.axis_index("c")
    base = wid * ROWS_PER_TILE

    pltpu.sync_copy(ts_hbm.at[wid], idx_v)
    for c in range(NCHUNK):
        for j in range(CHUNK // LANES):
            sl = pl.ds(j * LANES, LANES)
            v = idx_v[c, sl]
            idx_v[c, sl] = jnp.minimum(jnp.maximum(v, 0), MAX_LEN - 1)

    @pl.when(wid == 0)
    def _():
        pltpu.sync_copy(x_hbm.at[pl.ds(0, B)], xbuf.at[pl.ds(0, B)])
        pltpu.sync_copy(xbuf.at[pl.ds(0, B)], out_hbm.at[pl.ds(0, B)])

    def chunk_body(c, carry):
        row0 = B + base + c * CHUNK
        pltpu.async_copy(x_hbm.at[pl.ds(row0, CHUNK)], xbuf, sem_x).wait()
        pltpu.async_copy(pe_hbm.at[idx_v.at[c]], xbuf, sem_pe, add=True).wait()
        pltpu.sync_copy(xbuf, out_hbm.at[pl.ds(row0, CHUNK)])
        return carry

    lax.fori_loop(0, NCHUNK, chunk_body, 0)


def kernel(x, timestamps, pe):
    x2 = x.reshape((L + 1) * B, D_MODEL)
    ts3 = timestamps.T.reshape(NW, NCHUNK, CHUNK)
    pe2 = pe.reshape(MAX_LEN, D_MODEL)
    out = _pe_add(x2, ts3, pe2)
    return out.reshape(L + 1, B, D_MODEL)


# speedup vs baseline: 1.1434x; 1.0730x over previous
import functools

import jax
import jax.numpy as jnp
from jax import lax
from jax.experimental import pallas as pl
from jax.experimental.pallas import tpu as pltpu
from jax.experimental.pallas import tpu_sc as plsc

D_MODEL = 768
MAX_LEN = 8192
B = 4
L = 2048

NC = 2
NS = 16
NW = NC * NS
N_GATHER = B * L
ROWS_PER_TILE = N_GATHER // NW
CHUNK = 32
NCHUNK = ROWS_PER_TILE // CHUNK
LANES = 16
VECS_PER_ROW = D_MODEL // LANES
NBUF = 4
PRE = 2

_MESH = plsc.VectorSubcoreMesh(core_axis_name="c", subcore_axis_name="s")


@functools.partial(
    pl.kernel,
    out_type=jax.ShapeDtypeStruct((N_GATHER + B, D_MODEL), jnp.float32),
    mesh=_MESH,
    scratch_types=[
        pltpu.VMEM((NCHUNK, CHUNK), jnp.int32),
        [pltpu.VMEM((CHUNK, D_MODEL), jnp.float32) for _ in range(NBUF)],
        [pltpu.SemaphoreType.DMA for _ in range(NBUF)],
        [pltpu.SemaphoreType.DMA for _ in range(NBUF)],
        [pltpu.SemaphoreType.DMA for _ in range(NBUF)],
    ],
    compiler_params=pltpu.CompilerParams(use_tc_tiling_on_sc=False),
)
def _pe_add(x_hbm, ts_hbm, pe_hbm, out_hbm, idx_v, bufs, semx, semg, sems):
    wid = lax.axis_index("s") * NC + lax.axis_index("c")
    base = wid * ROWS_PER_TILE

    pltpu.sync_copy(ts_hbm.at[wid], idx_v)
    for c in range(NCHUNK):
        for j in range(CHUNK // LANES):
            sl = pl.ds(j * LANES, LANES)
            v = idx_v[c, sl]
            idx_v[c, sl] = jnp.minimum(jnp.maximum(v, 0), MAX_LEN - 1)

    @pl.when(wid == 0)
    def _():
        pltpu.sync_copy(x_hbm.at[pl.ds(0, B)], bufs[0].at[pl.ds(0, B)])
        pltpu.sync_copy(bufs[0].at[pl.ds(0, B)], out_hbm.at[pl.ds(0, B)])

    def row0(c):
        return B + base + c * CHUNK

    cp_x = [None] * NCHUNK
    cp_g = [None] * NCHUNK
    cp_s = [None] * NCHUNK
    for c in range(min(PRE, NCHUNK)):
        cp_x[c] = pltpu.async_copy(
            x_hbm.at[pl.ds(row0(c), CHUNK)], bufs[c % NBUF], semx[c % NBUF])
    for c in range(NCHUNK):
        b = c % NBUF
        cp_x[c].wait()
        cp_g[c] = pltpu.async_copy(pe_hbm.at[idx_v.at[c]], bufs[b], semg[b],
                                   add=True)
        cp_g[c].wait()
        cp_s[c] = pltpu.async_copy(bufs[b], out_hbm.at[pl.ds(row0(c), CHUNK)],
                                   sems[b])
        nxt = c + PRE
        if nxt < NCHUNK:
            if nxt - NBUF >= 0:
                cp_s[nxt - NBUF].wait()
            nb = nxt % NBUF
            cp_x[nxt] = pltpu.async_copy(
                x_hbm.at[pl.ds(row0(nxt), CHUNK)], bufs[nb], semx[nb])
    for c in range(max(0, NCHUNK - NBUF), NCHUNK):
        cp_s[c].wait()


def kernel(x, timestamps, pe):
    x2 = x.reshape((L + 1) * B, D_MODEL)
    ts3 = timestamps.T.reshape(NW, NCHUNK, CHUNK)
    pe2 = pe.reshape(MAX_LEN, D_MODEL)
    out = _pe_add(x2, ts3, pe2)
    return out.reshape(L + 1, B, D_MODEL)


# speedup vs baseline: 1.1637x; 1.0178x over previous
import functools

import jax
import jax.numpy as jnp
from jax import lax
from jax.experimental import pallas as pl
from jax.experimental.pallas import tpu as pltpu
from jax.experimental.pallas import tpu_sc as plsc

D_MODEL = 768
MAX_LEN = 8192
B = 4
L = 2048

NC = 2
NS = 16
NW = NC * NS
N_GATHER = B * L
ROWS_PER_TILE = N_GATHER // NW
CHUNK = 32
NCHUNK = ROWS_PER_TILE // CHUNK
LANES = 16
VECS_PER_ROW = D_MODEL // LANES
NBUF = 4
PRE = 2

_MESH = plsc.VectorSubcoreMesh(core_axis_name="c", subcore_axis_name="s")


@functools.partial(
    pl.kernel,
    out_type=jax.ShapeDtypeStruct((N_GATHER + B, D_MODEL), jnp.float32),
    mesh=_MESH,
    scratch_types=[
        pltpu.VMEM((NCHUNK, CHUNK), jnp.int32),
        [pltpu.VMEM((CHUNK, D_MODEL), jnp.float32) for _ in range(NBUF)],
        [pltpu.SemaphoreType.DMA for _ in range(NBUF)],
        [pltpu.SemaphoreType.DMA for _ in range(NBUF)],
        [pltpu.SemaphoreType.DMA for _ in range(NBUF)],
    ],
    compiler_params=pltpu.CompilerParams(use_tc_tiling_on_sc=False),
)
def _pe_add(x_hbm, ts_hbm, pe_hbm, out_hbm, idx_v, bufs, semx, semg, sems):
    wid = lax.axis_index("s") * NC + lax.axis_index("c")
    base = wid * ROWS_PER_TILE

    pltpu.sync_copy(ts_hbm.at[wid], idx_v)
    for c in range(NCHUNK):
        for j in range(CHUNK // LANES):
            sl = pl.ds(j * LANES, LANES)
            v = idx_v[c, sl]
            idx_v[c, sl] = jnp.minimum(jnp.maximum(v, 0), MAX_LEN - 1)

    @pl.when(wid == 0)
    def _():
        pltpu.sync_copy(x_hbm.at[pl.ds(0, B)], bufs[0].at[pl.ds(0, B)])
        pltpu.sync_copy(bufs[0].at[pl.ds(0, B)], out_hbm.at[pl.ds(0, B)])

    def row0(c):
        return B + base + c * CHUNK

    cp_x = [None] * NCHUNK
    cp_g = [None] * NCHUNK
    cp_s = [None] * NCHUNK
    for c in range(min(PRE, NCHUNK)):
        cp_x[c] = pltpu.async_copy(
            x_hbm.at[pl.ds(row0(c), CHUNK)], bufs[c % NBUF], semx[c % NBUF])
    for c in range(NCHUNK):
        b = c % NBUF
        cp_x[c].wait()
        cp_g[c] = pltpu.async_copy(pe_hbm.at[idx_v.at[c]], bufs[b], semg[b],
                                   add=True)
        if c > 0:
            cp_g[c - 1].wait()
            pb = (c - 1) % NBUF
            cp_s[c - 1] = pltpu.async_copy(
                bufs[pb], out_hbm.at[pl.ds(row0(c - 1), CHUNK)], sems[pb])
        nxt = c + PRE
        if nxt < NCHUNK:
            if nxt - NBUF >= 0:
                cp_s[nxt - NBUF].wait()
            nb = nxt % NBUF
            cp_x[nxt] = pltpu.async_copy(
                x_hbm.at[pl.ds(row0(nxt), CHUNK)], bufs[nb], semx[nb])
    last = NCHUNK - 1
    cp_g[last].wait()
    cp_s[last] = pltpu.async_copy(
        bufs[last % NBUF], out_hbm.at[pl.ds(row0(last), CHUNK)],
        sems[last % NBUF])
    for c in range(max(0, NCHUNK - NBUF), NCHUNK):
        cp_s[c].wait()


def kernel(x, timestamps, pe):
    x2 = x.reshape((L + 1) * B, D_MODEL)
    ts3 = timestamps.T.reshape(NW, NCHUNK, CHUNK)
    pe2 = pe.reshape(MAX_LEN, D_MODEL)
    out = _pe_add(x2, ts3, pe2)
    return out.reshape(L + 1, B, D_MODEL)
